# Initial kernel scaffold; baseline (speedup 1.0000x reference)
import jax
import jax.numpy as jnp
from jax.experimental import pallas as pl


def kernel(xyz):
    raise NotImplementedError("write your pallas kernel here")


# speedup vs baseline: 2.1955x; 2.1955x over previous
import numpy as np
import jax
import jax.numpy as jnp
from jax.experimental import pallas as pl
from jax.experimental.pallas import tpu as pltpu

NBINS = 100
R_START = 0.0
R_END = 7.0
BOX = 15.0
CUTOFF = R_END + 0.5
CUT2 = CUTOFF * CUTOFF
HALF = 0.5 * BOX

ROWS = 128
COLT = 512
BPAD = 104
PADV = 1.0e6

_offset64 = np.linspace(R_START, R_END, NBINS)
_width64 = _offset64[1] - _offset64[0]
_scale64 = np.sqrt(0.5) / _width64
OFF_SCALED = np.full((BPAD,), 1.0e4, dtype=np.float32)
OFF_SCALED[:NBINS] = (_scale64 * _offset64).astype(np.float32)
SCALE = np.float32(_scale64)

_bins64 = np.linspace(R_START, R_END, NBINS + 1)
_vol64 = 4.0 * np.pi / 3.0 * (_bins64[1:] ** 3 - _bins64[:-1] ** 3)
_V64 = 4.0 / 3.0 * np.pi * R_END ** 3
VFAC = (np.float64(_V64) / _vol64).astype(np.float32)
BINS = _bins64.astype(np.float32)


def _rdf_kernel(xyz_r_ref, xyz_c_ref, offs_ref, vfac_ref,
                count_ref, rdf_ref, acc_ref, dsc_ref, wsc_ref):
    i = pl.program_id(0)
    c = pl.program_id(1)
    ni = pl.num_programs(0)
    nc = pl.num_programs(1)

    @pl.when((i == 0) & (c == 0))
    def _init():
        acc_ref[...] = jnp.zeros_like(acc_ref)

    @pl.when((c + 1) * COLT > i * ROWS)
    def _compute():
        rowbase = i * ROWS
        colbase = c * COLT
        dsq = jnp.zeros((ROWS, COLT), jnp.float32)
        for dim in range(3):
            xr = xyz_r_ref[:, dim].reshape(ROWS, 1)
            xc = xyz_c_ref[dim, :].reshape(1, COLT)
            dx = xr - xc
            dx = jnp.where(dx >= HALF, dx - BOX, dx)
            dx = jnp.where(dx < -HALF, dx + BOX, dx)
            dsq = dsq + dx * dx
        row_ids = rowbase + jax.lax.broadcasted_iota(jnp.int32, (ROWS, COLT), 0)
        col_ids = colbase + jax.lax.broadcasted_iota(jnp.int32, (ROWS, COLT), 1)
        mask = (dsq < CUT2) & (dsq > 0.0) & (col_ids > row_ids)
        w = jnp.where(mask, 2.0, 0.0).astype(jnp.float32)
        ds = SCALE * jnp.sqrt(jnp.where(mask, dsq, 1.0))
        dsc_ref[...] = ds
        wsc_ref[...] = w

        def body(r, _):
            drb = jnp.broadcast_to(dsc_ref[pl.ds(r, 1), :], (8, COLT))
            wrb = jnp.broadcast_to(wsc_ref[pl.ds(r, 1), :], (8, COLT))
            for b in range(BPAD // 8):
                u = drb - offs_ref[8 * b:8 * b + 8, :]
                acc_ref[8 * b:8 * b + 8, :] += jnp.exp(-(u * u)) * wrb
            return 0

        jax.lax.fori_loop(0, ROWS, body, 0, unroll=2)

    @pl.when((i == ni - 1) & (c == nc - 1))
    def _finish():
        counts = jnp.sum(acc_ref[0:NBINS, :], axis=1).reshape(1, NBINS)
        norm = jnp.sum(counts)
        cn = counts / norm
        count_ref[...] = cn
        rdf_ref[...] = cn * vfac_ref[...]


def _rdf_call(xyz_pad, xyz_pad_t, offs, vfac):
    npad = xyz_pad.shape[0]
    grid = (npad // ROWS, npad // COLT)
    count, rdf = pl.pallas_call(
        _rdf_kernel,
        grid=grid,
        in_specs=[
            pl.BlockSpec((ROWS, 3), lambda i, c: (i, 0)),
            pl.BlockSpec((3, COLT), lambda i, c: (0, c)),
            pl.BlockSpec((BPAD, COLT), lambda i, c: (0, 0)),
            pl.BlockSpec((1, NBINS), lambda i, c: (0, 0)),
        ],
        out_specs=[
            pl.BlockSpec((1, NBINS), lambda i, c: (0, 0)),
            pl.BlockSpec((1, NBINS), lambda i, c: (0, 0)),
        ],
        out_shape=[
            jax.ShapeDtypeStruct((1, NBINS), jnp.float32),
            jax.ShapeDtypeStruct((1, NBINS), jnp.float32),
        ],
        scratch_shapes=[
            pltpu.VMEM((BPAD, COLT), jnp.float32),
            pltpu.VMEM((ROWS, COLT), jnp.float32),
            pltpu.VMEM((ROWS, COLT), jnp.float32),
        ],
    )(xyz_pad, xyz_pad_t, offs, vfac)
    return count, rdf


def kernel(xyz):
    n = xyz.shape[0]
    npad = ((n + COLT - 1) // COLT) * COLT
    xyz_pad = jnp.full((npad, 3), PADV, jnp.float32).at[:n, :].set(xyz)
    xyz_pad_t = xyz_pad.T
    offs = jnp.broadcast_to(jnp.asarray(OFF_SCALED)[:, None], (BPAD, COLT))
    vfac = jnp.asarray(VFAC).reshape(1, NBINS)
    count, rdf = _rdf_call(xyz_pad, xyz_pad_t, offs, vfac)
    bins = jnp.asarray(BINS)
    return count.reshape(NBINS), bins, rdf.reshape(NBINS)


# speedup vs baseline: 2.5211x; 1.1483x over previous
import numpy as np
import jax
import jax.numpy as jnp
from jax.experimental import pallas as pl
from jax.experimental.pallas import tpu as pltpu

NBINS = 100
R_START = 0.0
R_END = 7.0
BOX = 15.0
CUTOFF = R_END + 0.5
CUT2 = CUTOFF * CUTOFF
HALF = 0.5 * BOX

ROWS = 128
COLT = 512
BPAD = 104
PADV = 1.0e6

_offset64 = np.linspace(R_START, R_END, NBINS)
_width64 = _offset64[1] - _offset64[0]
_scale64 = np.sqrt(0.5) / _width64
_offs64 = np.full((BPAD,), 1.0e4, dtype=np.float64)
_offs64[:NBINS] = _scale64 * _offset64
OFF_A = (2.0 * _offs64).astype(np.float32)
OFF_C = (-(_offs64 ** 2)).astype(np.float32)
SCALE = np.float32(_scale64)
LOG2 = np.float32(np.log(2.0))

_bins64 = np.linspace(R_START, R_END, NBINS + 1)
_vol64 = 4.0 * np.pi / 3.0 * (_bins64[1:] ** 3 - _bins64[:-1] ** 3)
_V64 = 4.0 / 3.0 * np.pi * R_END ** 3
VFAC = (np.float64(_V64) / _vol64).astype(np.float32)
BINS = _bins64.astype(np.float32)


def _rdf_kernel(xyz_r_ref, xyz_c_ref, offa_ref, offc_ref, vfac_ref,
                count_ref, rdf_ref, acc_ref, dsc_ref, pwsc_ref):
    i = pl.program_id(0)
    c = pl.program_id(1)
    ni = pl.num_programs(0)
    nc = pl.num_programs(1)

    @pl.when((i == 0) & (c == 0))
    def _init():
        acc_ref[...] = jnp.zeros_like(acc_ref)

    @pl.when((c + 1) * COLT > i * ROWS)
    def _compute():
        rowbase = i * ROWS
        colbase = c * COLT
        dsq = jnp.zeros((ROWS, COLT), jnp.float32)
        for dim in range(3):
            xr = xyz_r_ref[:, dim].reshape(ROWS, 1)
            xc = xyz_c_ref[dim, :].reshape(1, COLT)
            dx = xr - xc
            dx = jnp.where(dx >= HALF, dx - BOX, dx)
            dx = jnp.where(dx < -HALF, dx + BOX, dx)
            dsq = dsq + dx * dx
        row_ids = rowbase + jax.lax.broadcasted_iota(jnp.int32, (ROWS, COLT), 0)
        col_ids = colbase + jax.lax.broadcasted_iota(jnp.int32, (ROWS, COLT), 1)
        mask = (dsq < CUT2) & (dsq > 0.0) & (col_ids > row_ids)
        ds = SCALE * jnp.sqrt(jnp.where(mask, dsq, 1.0))
        lw = jnp.where(mask, LOG2, -jnp.inf).astype(jnp.float32)
        dsc_ref[...] = ds
        pwsc_ref[...] = lw - ds * ds

        def body(r, _):
            drb = jnp.broadcast_to(dsc_ref[pl.ds(r, 1), :], (8, COLT))
            pwb = jnp.broadcast_to(pwsc_ref[pl.ds(r, 1), :], (8, COLT))
            for b in range(BPAD // 8):
                s = slice(8 * b, 8 * b + 8)
                acc_ref[s, :] += jnp.exp(offa_ref[s, :] * drb + (offc_ref[s, :] + pwb))
            return 0

        jax.lax.fori_loop(0, ROWS, body, 0, unroll=2)

    @pl.when((i == ni - 1) & (c == nc - 1))
    def _finish():
        counts = jnp.sum(acc_ref[0:NBINS, :], axis=1).reshape(1, NBINS)
        norm = jnp.sum(counts)
        cn = counts / norm
        count_ref[...] = cn
        rdf_ref[...] = cn * vfac_ref[...]


def _rdf_call(xyz_pad, xyz_pad_t, offa, offc, vfac):
    npad = xyz_pad.shape[0]
    grid = (npad // ROWS, npad // COLT)
    count, rdf = pl.pallas_call(
        _rdf_kernel,
        grid=grid,
        in_specs=[
            pl.BlockSpec((ROWS, 3), lambda i, c: (i, 0)),
            pl.BlockSpec((3, COLT), lambda i, c: (0, c)),
            pl.BlockSpec((BPAD, COLT), lambda i, c: (0, 0)),
            pl.BlockSpec((BPAD, COLT), lambda i, c: (0, 0)),
            pl.BlockSpec((1, NBINS), lambda i, c: (0, 0)),
        ],
        out_specs=[
            pl.BlockSpec((1, NBINS), lambda i, c: (0, 0)),
            pl.BlockSpec((1, NBINS), lambda i, c: (0, 0)),
        ],
        out_shape=[
            jax.ShapeDtypeStruct((1, NBINS), jnp.float32),
            jax.ShapeDtypeStruct((1, NBINS), jnp.float32),
        ],
        scratch_shapes=[
            pltpu.VMEM((BPAD, COLT), jnp.float32),
            pltpu.VMEM((ROWS, COLT), jnp.float32),
            pltpu.VMEM((ROWS, COLT), jnp.float32),
        ],
    )(xyz_pad, xyz_pad_t, offa, offc, vfac)
    return count, rdf


def kernel(xyz):
    n = xyz.shape[0]
    npad = ((n + COLT - 1) // COLT) * COLT
    xyz_pad = jnp.full((npad, 3), PADV, jnp.float32).at[:n, :].set(xyz)
    xyz_pad_t = xyz_pad.T
    offa = jnp.broadcast_to(jnp.asarray(OFF_A)[:, None], (BPAD, COLT))
    offc = jnp.broadcast_to(jnp.asarray(OFF_C)[:, None], (BPAD, COLT))
    vfac = jnp.asarray(VFAC).reshape(1, NBINS)
    count, rdf = _rdf_call(xyz_pad, xyz_pad_t, offa, offc, vfac)
    bins = jnp.asarray(BINS)
    return count.reshape(NBINS), bins, rdf.reshape(NBINS)


# speedup vs baseline: 2.9246x; 1.1601x over previous
import numpy as np
import jax
import jax.numpy as jnp
from jax.experimental import pallas as pl
from jax.experimental.pallas import tpu as pltpu

NBINS = 100
R_START = 0.0
R_END = 7.0
BOX = 15.0
CUTOFF = R_END + 0.5
CUT2 = CUTOFF * CUTOFF
HALF = 0.5 * BOX

ROWS = 128
COLT = 512
BPAD = 104
PADV = 1.0e6

_offset64 = np.linspace(R_START, R_END, NBINS)
_width64 = _offset64[1] - _offset64[0]
_sl64 = np.sqrt(0.5 * np.log2(np.e)) / _width64
_offl64 = np.full((BPAD,), 1.0e4, dtype=np.float64)
_offl64[:NBINS] = _sl64 * _offset64
OFF_L = _offl64.astype(np.float32)
SCALE_L = np.float32(_sl64)

_bins64 = np.linspace(R_START, R_END, NBINS + 1)
_vol64 = 4.0 * np.pi / 3.0 * (_bins64[1:] ** 3 - _bins64[:-1] ** 3)
_V64 = 4.0 / 3.0 * np.pi * R_END ** 3
VFAC = (np.float64(_V64) / _vol64).astype(np.float32)
BINS = _bins64.astype(np.float32)


def _hist_kernel(xyz_r_ref, xyz_c_ref, offl_ref, acc_ref, dsc_ref, lwsc_ref):
    i = pl.program_id(0)
    c = pl.program_id(1)

    @pl.when((i == 0) & (c == 0))
    def _init():
        acc_ref[...] = jnp.zeros_like(acc_ref)

    @pl.when((c + 1) * COLT > i * ROWS)
    def _compute():
        dsq = jnp.zeros((ROWS, COLT), jnp.float32)
        for dim in range(3):
            xr = xyz_r_ref[:, dim].reshape(ROWS, 1)
            xc = xyz_c_ref[dim, :].reshape(1, COLT)
            dx = xr - xc
            dx = jnp.where(dx >= HALF, dx - BOX, dx)
            dx = jnp.where(dx < -HALF, dx + BOX, dx)
            dsq = dsq + dx * dx
        row_ids = i * ROWS + jax.lax.broadcasted_iota(jnp.int32, (ROWS, COLT), 0)
        col_ids = c * COLT + jax.lax.broadcasted_iota(jnp.int32, (ROWS, COLT), 1)
        mask = (dsq < CUT2) & (dsq > 0.0) & (col_ids > row_ids)
        dsc_ref[...] = SCALE_L * jnp.sqrt(jnp.where(mask, dsq, 1.0))
        lwsc_ref[...] = jnp.where(mask, 1.0, -jnp.inf).astype(jnp.float32)

        def body(r, _):
            drb = jnp.broadcast_to(dsc_ref[pl.ds(r, 1), :], (8, COLT))
            lwb = jnp.broadcast_to(lwsc_ref[pl.ds(r, 1), :], (8, COLT))
            for b in range(BPAD // 8):
                s = slice(8 * b, 8 * b + 8)
                u = drb - offl_ref[s, :]
                acc_ref[s, :] += jnp.exp2(lwb - u * u)
            return 0

        jax.lax.fori_loop(0, ROWS, body, 0, unroll=2)


def _fin_kernel(acc_ref, vfac_ref, count_ref, rdf_ref):
    counts = jnp.sum(acc_ref[0:NBINS, :], axis=1).reshape(1, NBINS)
    norm = jnp.sum(counts)
    cn = counts / norm
    count_ref[...] = cn
    rdf_ref[...] = cn * vfac_ref[...]


def _rdf_call(xyz_pad, xyz_pad_t, offl, vfac):
    npad = xyz_pad.shape[0]
    grid = (npad // ROWS, npad // COLT)
    acc = pl.pallas_call(
        _hist_kernel,
        grid=grid,
        in_specs=[
            pl.BlockSpec((ROWS, 3), lambda i, c: (i, 0)),
            pl.BlockSpec((3, COLT), lambda i, c: (0, c)),
            pl.BlockSpec((BPAD, COLT), lambda i, c: (0, 0)),
        ],
        out_specs=pl.BlockSpec((BPAD, COLT), lambda i, c: (0, 0)),
        out_shape=jax.ShapeDtypeStruct((BPAD, COLT), jnp.float32),
        scratch_shapes=[
            pltpu.VMEM((ROWS, COLT), jnp.float32),
            pltpu.VMEM((ROWS, COLT), jnp.float32),
        ],
    )(xyz_pad, xyz_pad_t, offl)
    count, rdf = pl.pallas_call(
        _fin_kernel,
        out_shape=[
            jax.ShapeDtypeStruct((1, NBINS), jnp.float32),
            jax.ShapeDtypeStruct((1, NBINS), jnp.float32),
        ],
    )(acc, vfac)
    return count, rdf


def kernel(xyz):
    n = xyz.shape[0]
    npad = ((n + COLT - 1) // COLT) * COLT
    xyz_pad = jnp.full((npad, 3), PADV, jnp.float32).at[:n, :].set(xyz)
    xyz_pad_t = xyz_pad.T
    offl = jnp.broadcast_to(jnp.asarray(OFF_L)[:, None], (BPAD, COLT))
    vfac = jnp.asarray(VFAC).reshape(1, NBINS)
    count, rdf = _rdf_call(xyz_pad, xyz_pad_t, offl, vfac)
    bins = jnp.asarray(BINS)
    return count.reshape(NBINS), bins, rdf.reshape(NBINS)


# speedup vs baseline: 3.1103x; 1.0635x over previous
import numpy as np
import jax
import jax.numpy as jnp
from jax.experimental import pallas as pl
from jax.experimental.pallas import tpu as pltpu

NBINS = 100
R_START = 0.0
R_END = 7.0
BOX = 15.0
CUTOFF = R_END + 0.5
CUT2 = CUTOFF * CUTOFF
HALF = 0.5 * BOX

ROWS = 128
COLT = 512
BPAD = 104
PADV = 1.0e6

_offset64 = np.linspace(R_START, R_END, NBINS)
_width64 = _offset64[1] - _offset64[0]
_sl64 = np.sqrt(0.5 * np.log2(np.e)) / _width64
_offl64 = np.full((BPAD,), 1.0e4, dtype=np.float64)
_offl64[:NBINS] = _sl64 * _offset64
OFF_L = _offl64.astype(np.float32)
SCALE_L = np.float32(_sl64)

_bins64 = np.linspace(R_START, R_END, NBINS + 1)
_vol64 = 4.0 * np.pi / 3.0 * (_bins64[1:] ** 3 - _bins64[:-1] ** 3)
_V64 = 4.0 / 3.0 * np.pi * R_END ** 3
VFAC = (np.float64(_V64) / _vol64).astype(np.float32)
BINS = _bins64.astype(np.float32)


def _hist_kernel(xyz_r_ref, xyz_c_ref, offl_ref, acc_ref, dsc_ref):
    i = pl.program_id(0)
    c = pl.program_id(1)

    @pl.when((i == 0) & (c == 0))
    def _init():
        acc_ref[...] = jnp.zeros_like(acc_ref)

    @pl.when((c + 1) * COLT > i * ROWS)
    def _compute():
        dsq = jnp.zeros((ROWS, COLT), jnp.float32)
        for dim in range(3):
            xr = xyz_r_ref[:, dim].reshape(ROWS, 1)
            xc = xyz_c_ref[dim, :].reshape(1, COLT)
            t = jnp.abs(xr - xc)
            t = jnp.minimum(t, BOX - t)
            dsq = dsq + t * t
        row_ids = i * ROWS + jax.lax.broadcasted_iota(jnp.int32, (ROWS, COLT), 0)
        col_ids = c * COLT + jax.lax.broadcasted_iota(jnp.int32, (ROWS, COLT), 1)
        mask = (dsq < CUT2) & (dsq > 0.0) & (col_ids > row_ids)
        dsc_ref[...] = jnp.where(mask, SCALE_L * jnp.sqrt(dsq), 3.0e4)

        def body(r, _):
            drb = jnp.broadcast_to(dsc_ref[pl.ds(r, 1), :], (8, COLT))
            for b in range(BPAD // 8):
                s = slice(8 * b, 8 * b + 8)
                u = drb - offl_ref[s, :]
                acc_ref[s, :] += jnp.exp2(-(u * u))
            return 0

        jax.lax.fori_loop(0, ROWS, body, 0, unroll=4)


def _fin_kernel(acc_ref, vfac_ref, count_ref, rdf_ref):
    counts = jnp.sum(acc_ref[0:NBINS, :], axis=1).reshape(1, NBINS)
    norm = jnp.sum(counts)
    cn = counts / norm
    count_ref[...] = cn
    rdf_ref[...] = cn * vfac_ref[...]


def _rdf_call(xyz_pad, xyz_pad_t, offl, vfac):
    npad = xyz_pad.shape[0]
    grid = (npad // ROWS, npad // COLT)
    acc = pl.pallas_call(
        _hist_kernel,
        grid=grid,
        in_specs=[
            pl.BlockSpec((ROWS, 3), lambda i, c: (i, 0)),
            pl.BlockSpec((3, COLT), lambda i, c: (0, c)),
            pl.BlockSpec((BPAD, COLT), lambda i, c: (0, 0)),
        ],
        out_specs=pl.BlockSpec((BPAD, COLT), lambda i, c: (0, 0)),
        out_shape=jax.ShapeDtypeStruct((BPAD, COLT), jnp.float32),
        scratch_shapes=[
            pltpu.VMEM((ROWS, COLT), jnp.float32),
        ],
    )(xyz_pad, xyz_pad_t, offl)
    count, rdf = pl.pallas_call(
        _fin_kernel,
        out_shape=[
            jax.ShapeDtypeStruct((1, NBINS), jnp.float32),
            jax.ShapeDtypeStruct((1, NBINS), jnp.float32),
        ],
    )(acc, vfac)
    return count, rdf


def kernel(xyz):
    n = xyz.shape[0]
    npad = ((n + COLT - 1) // COLT) * COLT
    xyz_pad = jnp.full((npad, 3), PADV, jnp.float32).at[:n, :].set(xyz)
    xyz_pad_t = xyz_pad.T
    offl = jnp.broadcast_to(jnp.asarray(OFF_L)[:, None], (BPAD, COLT))
    vfac = jnp.asarray(VFAC).reshape(1, NBINS)
    count, rdf = _rdf_call(xyz_pad, xyz_pad_t, offl, vfac)
    bins = jnp.asarray(BINS)
    return count.reshape(NBINS), bins, rdf.reshape(NBINS)
